# baseline (speedup 1.0000x reference)
import functools

import jax
import jax.numpy as jnp
from jax import lax
from jax.experimental import pallas as pl
from jax.experimental.pallas import tpu as pltpu
from jax.experimental.pallas import tpu_sc as plsc

_INFO = plsc.get_sparse_core_info()
_NC, _NS, _L = _INFO.num_cores, _INFO.num_subcores, _INFO.num_lanes
_NW = _NC * _NS

_B = 4
_S = 8192
_D = 64
_DP = 128
_N = _B * _S
_BPW = _N // _NW
_WPB = _S // _BPW
_CHUNK = 256
_GCH = 128
_NGR = _CHUNK // _GCH

_mesh = plsc.VectorSubcoreMesh(core_axis_name="c", subcore_axis_name="s")


@functools.partial(
    pl.kernel,
    mesh=_mesh,
    compiler_params=pltpu.CompilerParams(use_tc_tiling_on_sc=False),
    out_type=jax.ShapeDtypeStruct((_B, _S, _D), jnp.float32),
    scratch_types=[
        pltpu.VMEM((_BPW,), jnp.int32),
        pltpu.VMEM((_BPW,), jnp.int32),
        pltpu.VMEM((_CHUNK, _DP), jnp.float32),
        pltpu.VMEM((_CHUNK, _DP), jnp.float32),
        pltpu.SemaphoreType.DMA,
        pltpu.SemaphoreType.DMA,
        pltpu.SemaphoreType.DMA,
        pltpu.SemaphoreType.DMA,
    ],
)
def _sc_embed(w_hbm, idx_hbm, mask_hbm, out_hbm,
              idx_v, mask_v, rows_a, rows_b, sem_a, sem_b, osem_a, osem_b):
    wid = lax.axis_index("s") * _NC + lax.axis_index("c")
    b = wid // _WPB
    s0 = pl.multiple_of((wid % _WPB) * _BPW, _BPW)
    base = wid * _BPW

    pltpu.sync_copy(idx_hbm.at[pl.ds(base, _BPW)], idx_v)
    pltpu.sync_copy(mask_hbm.at[pl.ds(base, _BPW)], mask_v)

    lane = lax.iota(jnp.int32, _L)
    half = jnp.full((_L,), 0.5, jnp.float32)

    for g in range(_BPW // _L):
        sl = pl.ds(g * _L, _L)
        pos = base + g * _L + lane
        idx_v[sl] = jnp.where(mask_v[sl] != 0, pos, idx_v[sl])

    bufs = (rows_a, rows_b)
    sems = (sem_a, sem_b)
    nchunks = _BPW // _CHUNK

    def fire(c):
        buf, sm = bufs[c % 2], sems[c % 2]
        return [
            pltpu.async_copy(
                w_hbm.at[idx_v.at[pl.ds(c * _CHUNK + j * _GCH, _GCH)]],
                buf.at[pl.ds(j * _GCH, _GCH)],
                sm,
            )
            for j in range(_NGR)
        ]

    osems = (osem_a, osem_b)
    inflight = fire(0)
    outflight = [None, None]
    for c in range(nchunks):
        c0 = c * _CHUNK
        buf = bufs[c % 2]
        if c + 1 < nchunks:
            prev_out = outflight[(c + 1) % 2]
            if prev_out is not None:
                prev_out.wait()
                outflight[(c + 1) % 2] = None
            nxt = fire(c + 1)
        else:
            nxt = []
        for cp in inflight:
            cp.wait()
        inflight = nxt

        def fix_group(g, carry, c0=c0, buf=buf):
            m = mask_v[pl.ds(c0 + g * _L, _L)]
            for l in range(_L):
                @pl.when(m[l] != 0)
                def _():
                    r = g * _L + l
                    for cc in range(_D // _L):
                        buf[r, pl.ds(cc * _L, _L)] = half
            return carry

        lax.fori_loop(0, _CHUNK // _L, fix_group, 0)

        outflight[c % 2] = pltpu.async_copy(
            buf.at[:, pl.ds(0, _D)],
            out_hbm.at[b, pl.ds(s0 + c0, _CHUNK)],
            osems[c % 2],
        )

    for oc in outflight:
        if oc is not None:
            oc.wait()


def kernel(input_ids, pixel_values, grid_thw, img_mask, W):
    del pixel_values, grid_thw
    idx = input_ids.reshape(-1)
    mask = img_mask.reshape(-1).astype(jnp.int32)
    w_pad = jnp.pad(W, ((0, 0), (0, _DP - _D)))
    return _sc_embed(w_pad, idx, mask)


# speedup vs baseline: 1.0026x; 1.0026x over previous
import functools

import jax
import jax.numpy as jnp
from jax import lax
from jax.experimental import pallas as pl
from jax.experimental.pallas import tpu as pltpu
from jax.experimental.pallas import tpu_sc as plsc

_INFO = plsc.get_sparse_core_info()
_NC, _NS, _L = _INFO.num_cores, _INFO.num_subcores, _INFO.num_lanes
_NW = _NC * _NS

_B = 4
_S = 8192
_D = 64
_DP = 128
_N = _B * _S
_BPW = _N // _NW
_WPB = _S // _BPW
_CHUNK = 256
_GCH = 128
_NGR = _CHUNK // _GCH

_mesh = plsc.VectorSubcoreMesh(core_axis_name="c", subcore_axis_name="s")


@functools.partial(
    pl.kernel,
    mesh=_mesh,
    compiler_params=pltpu.CompilerParams(use_tc_tiling_on_sc=False),
    out_type=jax.ShapeDtypeStruct((_B, _S, _D), jnp.float32),
    scratch_types=[
        pltpu.VMEM((_BPW,), jnp.int32),
        pltpu.VMEM((_BPW,), jnp.int32),
        pltpu.VMEM((_CHUNK, _DP), jnp.float32),
        pltpu.VMEM((_CHUNK, _DP), jnp.float32),
        pltpu.SemaphoreType.DMA,
        pltpu.SemaphoreType.DMA,
    ],
)
def _sc_embed(w_hbm, idx_hbm, mask_hbm, out_hbm,
              idx_v, mask_v, rows_a, rows_b, sem_a, sem_b):
    wid = lax.axis_index("s") * _NC + lax.axis_index("c")
    b = wid // _WPB
    s0 = pl.multiple_of((wid % _WPB) * _BPW, _BPW)
    base = wid * _BPW

    pltpu.sync_copy(idx_hbm.at[pl.ds(base, _BPW)], idx_v)
    pltpu.sync_copy(mask_hbm.at[pl.ds(base, _BPW)], mask_v)

    lane = lax.iota(jnp.int32, _L)
    half = jnp.full((_L,), 0.5, jnp.float32)

    for g in range(_BPW // _L):
        sl = pl.ds(g * _L, _L)
        pos = base + g * _L + lane
        idx_v[sl] = jnp.where(mask_v[sl] != 0, pos, idx_v[sl])

    bufs = (rows_a, rows_b)
    sems = (sem_a, sem_b)
    nchunks = _BPW // _CHUNK

    def fire(c):
        buf, sm = bufs[c % 2], sems[c % 2]
        return [
            pltpu.async_copy(
                w_hbm.at[idx_v.at[pl.ds(c * _CHUNK + j * _GCH, _GCH)]],
                buf.at[pl.ds(j * _GCH, _GCH)],
                sm,
            )
            for j in range(_NGR)
        ]

    inflight = fire(0)
    for c in range(nchunks):
        c0 = c * _CHUNK
        buf = bufs[c % 2]
        nxt = fire(c + 1) if c + 1 < nchunks else []
        for cp in inflight:
            cp.wait()
        inflight = nxt

        def fix_group(g, carry, c0=c0, buf=buf):
            m = mask_v[pl.ds(c0 + g * _L, _L)]
            for l in range(_L):
                @pl.when(m[l] != 0)
                def _():
                    r = g * _L + l
                    for cc in range(_D // _L):
                        buf[r, pl.ds(cc * _L, _L)] = half
            return carry

        lax.fori_loop(0, _CHUNK // _L, fix_group, 0)

        pltpu.sync_copy(
            buf.at[:, pl.ds(0, _D)],
            out_hbm.at[b, pl.ds(s0 + c0, _CHUNK)],
        )


def kernel(input_ids, pixel_values, grid_thw, img_mask, W):
    del pixel_values, grid_thw
    idx = input_ids.reshape(-1)
    mask = img_mask.reshape(-1).astype(jnp.int32)
    w_pad = jnp.pad(W, ((0, 0), (0, _DP - _D)))
    return _sc_embed(w_pad, idx, mask)
